# baseline (speedup 1.0000x reference)
import functools

import jax
import jax.numpy as jnp
from jax import lax
from jax.experimental import pallas as pl
from jax.experimental.pallas import tpu as pltpu
from jax.experimental.pallas import tpu_sc as plsc

N = 10000
E = 320000
D = 128

NC = 2
NS = 16
NW = NC * NS
CHUNK = 128
NCH = E // CHUNK
CH_BASE = NCH // NW
CH_EXTRA = NCH % NW
MAXCH = CH_BASE + 1
NPAD = 10240
WIN = 16


_MB = 512


def _mv_body(x_ref, w_ref, o_ref):
    res = jax.lax.dot_general(
        x_ref[...], w_ref[...], (((1,), (0,)), ((), ())),
        preferred_element_type=jnp.float32)
    o_ref[...] = jnp.transpose(res, (1, 0))


def _support(x, W):
    grid = -(-N // _MB)
    return pl.pallas_call(
        _mv_body,
        grid=(grid,),
        in_specs=[
            pl.BlockSpec((_MB, D), lambda i: (i, 0)),
            pl.BlockSpec((D, 1), lambda i: (0, 0)),
        ],
        out_specs=pl.BlockSpec((1, _MB), lambda i: (0, i)),
        out_shape=jax.ShapeDtypeStruct((1, N), jnp.float32),
    )(x, W)


def _sc_edge_body(support_hbm, eidx_hbm, ew_hbm, out_hbm,
                  support_v, src_v, dst_v, ew_v, msgs_v, zbuf_v, agg_sh,
                  sem_sup, sem_src, sem_dst, sem_ew, sem_scat):
    cid = lax.axis_index("c")
    sid = lax.axis_index("s")
    wid = sid * NC + cid

    base = wid * CH_BASE + jnp.minimum(wid, CH_EXTRA)
    has_extra = wid < CH_EXTRA
    count = CH_BASE + has_extra.astype(jnp.int32)

    c_sup = pltpu.async_copy(support_hbm.at[0], support_v, sem_sup)
    c_src = pltpu.async_copy(eidx_hbm.at[0, pl.ds(base, CH_BASE), :],
                             src_v.at[pl.ds(0, CH_BASE), :], sem_src)
    c_dst = pltpu.async_copy(eidx_hbm.at[1, pl.ds(base, CH_BASE), :],
                             dst_v.at[pl.ds(0, CH_BASE), :], sem_dst)
    c_ew = pltpu.async_copy(ew_hbm.at[pl.ds(base * CHUNK, CH_BASE * CHUNK)],
                            ew_v.at[pl.ds(0, CH_BASE * CHUNK)], sem_ew)

    @pl.when(has_extra)
    def _():
        pltpu.async_copy(eidx_hbm.at[0, pl.ds(base + CH_BASE, 1), :],
                         src_v.at[pl.ds(CH_BASE, 1), :], sem_src)
        pltpu.async_copy(eidx_hbm.at[1, pl.ds(base + CH_BASE, 1), :],
                         dst_v.at[pl.ds(CH_BASE, 1), :], sem_dst)
        pltpu.async_copy(
            ew_hbm.at[pl.ds((base + CH_BASE) * CHUNK, CHUNK)],
            ew_v.at[pl.ds(CH_BASE * CHUNK, CHUNK)], sem_ew)

    def _z(i, c):
        zbuf_v[pl.ds(i * 16, 16)] = jnp.zeros((16,), jnp.float32)
        return c
    lax.fori_loop(0, NPAD // NS // 16, _z, 0, unroll=True)
    pltpu.sync_copy(zbuf_v, agg_sh.at[pl.ds(sid * (NPAD // NS), NPAD // NS)])

    c_sup.wait()
    c_src.wait()
    c_dst.wait()
    c_ew.wait()

    @pl.when(has_extra)
    def _():
        pltpu.make_async_copy(eidx_hbm.at[0, pl.ds(base + CH_BASE, 1), :],
                              src_v.at[pl.ds(CH_BASE, 1), :], sem_src).wait()
        pltpu.make_async_copy(eidx_hbm.at[1, pl.ds(base + CH_BASE, 1), :],
                              dst_v.at[pl.ds(CH_BASE, 1), :], sem_dst).wait()
        pltpu.make_async_copy(
            ew_hbm.at[pl.ds((base + CH_BASE) * CHUNK, CHUNK)],
            ew_v.at[pl.ds(CH_BASE * CHUNK, CHUNK)], sem_ew).wait()

    plsc.subcore_barrier()

    def _chunk(j, carry):
        for g in range(CHUNK // 16):
            s16 = src_v[j, pl.ds(g * 16, 16)]
            vals = plsc.load_gather(support_v, [s16])
            w16 = ew_v[pl.ds(j * CHUNK + g * 16, 16)]
            msgs_v[j, pl.ds(g * 16, 16)] = vals * w16
        pltpu.async_copy(msgs_v.at[j], agg_sh.at[dst_v.at[j]], sem_scat,
                         add=True)

        @pl.when(j >= WIN)
        def _():
            pltpu.make_async_copy(msgs_v.at[0], agg_sh.at[dst_v.at[0]],
                                  sem_scat).wait()
        return carry

    lax.fori_loop(0, count, _chunk, 0)

    def _drain(j, carry):
        pltpu.make_async_copy(msgs_v.at[0], agg_sh.at[dst_v.at[0]],
                              sem_scat).wait()
        return carry
    lax.fori_loop(0, jnp.minimum(count, WIN), _drain, 0)

    plsc.subcore_barrier()

    sl = NPAD // NS
    pltpu.sync_copy(agg_sh.at[pl.ds(sid * sl, sl)],
                    out_hbm.at[cid, pl.ds(sid * sl, sl)])


_sc_edge = functools.partial(
    pl.kernel,
    out_type=jax.ShapeDtypeStruct((NC, NPAD), jnp.float32),
    mesh=plsc.VectorSubcoreMesh(core_axis_name="c", subcore_axis_name="s"),
    scratch_types=[
        pltpu.VMEM((N,), jnp.float32),
        pltpu.VMEM((MAXCH, CHUNK), jnp.int32),
        pltpu.VMEM((MAXCH, CHUNK), jnp.int32),
        pltpu.VMEM((MAXCH * CHUNK,), jnp.float32),
        pltpu.VMEM((MAXCH, CHUNK), jnp.float32),
        pltpu.VMEM((NPAD // NS,), jnp.float32),
        pltpu.VMEM_SHARED((NPAD,), jnp.float32),
        pltpu.SemaphoreType.DMA,
        pltpu.SemaphoreType.DMA,
        pltpu.SemaphoreType.DMA,
        pltpu.SemaphoreType.DMA,
        pltpu.SemaphoreType.DMA,
    ],
    compiler_params=pltpu.CompilerParams(needs_layout_passes=False,
                                         use_tc_tiling_on_sc=False),
)(_sc_edge_body)


_RB = 1024
_RT = _RB // D


def _scale_body(x_ref, agg_ref, b_ref, eye_ref, o_ref):
    a = agg_ref[0] + agg_ref[1] + b_ref[0]
    attn = jnp.tanh(a)
    t = jax.lax.dot_general(eye_ref[...], attn, (((1,), (1,)), ((), ())),
                            preferred_element_type=jnp.float32,
                            precision=jax.lax.Precision.HIGHEST)
    for r in range(_RT):
        o_ref[pl.ds(r * D, D), :] = (
            x_ref[pl.ds(r * D, D), :] * t[:, r:r + 1])


def _scale(x, agg3, b, eye):
    grid = -(-N // _RB)
    return pl.pallas_call(
        _scale_body,
        grid=(grid,),
        in_specs=[
            pl.BlockSpec((_RB, D), lambda i: (i, 0)),
            pl.BlockSpec((NC, _RT, D), lambda i: (0, i, 0)),
            pl.BlockSpec(memory_space=pltpu.SMEM),
            pl.BlockSpec((D, D), lambda i: (0, 0)),
        ],
        out_specs=pl.BlockSpec((_RB, D), lambda i: (i, 0)),
        out_shape=jax.ShapeDtypeStruct((N, D), jnp.float32),
    )(x, agg3, b, eye)


def kernel(x, edge_index, edge_weight, W, b):
    eidx3 = edge_index.astype(jnp.int32).reshape(2, NCH, CHUNK)
    support = _support(x, W)
    agg2 = _sc_edge(support, eidx3, edge_weight)
    agg3 = agg2.reshape(NC, NPAD // D, D)
    return _scale(x, agg3, b, jnp.eye(D, dtype=jnp.float32))


# speedup vs baseline: 1.2179x; 1.2179x over previous
import functools

import jax
import jax.numpy as jnp
from jax import lax
from jax.experimental import pallas as pl
from jax.experimental.pallas import tpu as pltpu
from jax.experimental.pallas import tpu_sc as plsc

N = 10000
E = 320000
D = 128

NC = 2
NS = 16
NW = NC * NS
CHUNK = 128
NCH = E // CHUNK
CH_BASE = NCH // NW
CH_EXTRA = NCH % NW
MAXCH = CH_BASE + 1
NPAD = 10240
WIN = 16


_MB = 512


def _mv_body(wt_ref, x_ref, o_ref):
    o_ref[...] = jax.lax.dot_general(
        wt_ref[...], x_ref[...], (((1,), (1,)), ((), ())),
        preferred_element_type=jnp.float32)


def _support(x, Wt):
    return pl.pallas_call(
        _mv_body,
        out_shape=jax.ShapeDtypeStruct((1, N), jnp.float32),
    )(Wt, x)


def _sc_edge_body(support_hbm, eidx_hbm, ew_hbm, out_hbm,
                  support_v, src_v, dst_v, ew_v, msgs_v, zbuf_v, agg_sh,
                  sem_sup, sem_src, sem_dst, sem_ew, sem_scat):
    cid = lax.axis_index("c")
    sid = lax.axis_index("s")
    wid = sid * NC + cid

    base = wid * CH_BASE + jnp.minimum(wid, CH_EXTRA)
    has_extra = wid < CH_EXTRA
    count = CH_BASE + has_extra.astype(jnp.int32)

    c_sup = pltpu.async_copy(support_hbm.at[0], support_v, sem_sup)
    c_src = pltpu.async_copy(eidx_hbm.at[0, pl.ds(base, CH_BASE), :],
                             src_v.at[pl.ds(0, CH_BASE), :], sem_src)
    c_dst = pltpu.async_copy(eidx_hbm.at[1, pl.ds(base, CH_BASE), :],
                             dst_v.at[pl.ds(0, CH_BASE), :], sem_dst)
    c_ew = pltpu.async_copy(ew_hbm.at[pl.ds(base * CHUNK, CH_BASE * CHUNK)],
                            ew_v.at[pl.ds(0, CH_BASE * CHUNK)], sem_ew)

    @pl.when(has_extra)
    def _():
        pltpu.async_copy(eidx_hbm.at[0, pl.ds(base + CH_BASE, 1), :],
                         src_v.at[pl.ds(CH_BASE, 1), :], sem_src)
        pltpu.async_copy(eidx_hbm.at[1, pl.ds(base + CH_BASE, 1), :],
                         dst_v.at[pl.ds(CH_BASE, 1), :], sem_dst)
        pltpu.async_copy(
            ew_hbm.at[pl.ds((base + CH_BASE) * CHUNK, CHUNK)],
            ew_v.at[pl.ds(CH_BASE * CHUNK, CHUNK)], sem_ew)

    def _z(i, c):
        zbuf_v[pl.ds(i * 16, 16)] = jnp.zeros((16,), jnp.float32)
        return c
    lax.fori_loop(0, NPAD // NS // 16, _z, 0, unroll=True)
    pltpu.sync_copy(zbuf_v, agg_sh.at[pl.ds(sid * (NPAD // NS), NPAD // NS)])

    c_sup.wait()
    c_src.wait()
    c_dst.wait()
    c_ew.wait()

    @pl.when(has_extra)
    def _():
        pltpu.make_async_copy(eidx_hbm.at[0, pl.ds(base + CH_BASE, 1), :],
                              src_v.at[pl.ds(CH_BASE, 1), :], sem_src).wait()
        pltpu.make_async_copy(eidx_hbm.at[1, pl.ds(base + CH_BASE, 1), :],
                              dst_v.at[pl.ds(CH_BASE, 1), :], sem_dst).wait()
        pltpu.make_async_copy(
            ew_hbm.at[pl.ds((base + CH_BASE) * CHUNK, CHUNK)],
            ew_v.at[pl.ds(CH_BASE * CHUNK, CHUNK)], sem_ew).wait()

    plsc.subcore_barrier()

    def _chunk(j, carry):
        for g in range(CHUNK // 16):
            s16 = src_v[j, pl.ds(g * 16, 16)]
            vals = plsc.load_gather(support_v, [s16])
            w16 = ew_v[pl.ds(j * CHUNK + g * 16, 16)]
            msgs_v[j, pl.ds(g * 16, 16)] = vals * w16
        pltpu.async_copy(msgs_v.at[j], agg_sh.at[dst_v.at[j]], sem_scat,
                         add=True)

        @pl.when(j >= WIN)
        def _():
            pltpu.make_async_copy(msgs_v.at[0], agg_sh.at[dst_v.at[0]],
                                  sem_scat).wait()
        return carry

    lax.fori_loop(0, count, _chunk, 0)

    def _drain(j, carry):
        pltpu.make_async_copy(msgs_v.at[0], agg_sh.at[dst_v.at[0]],
                              sem_scat).wait()
        return carry
    lax.fori_loop(0, jnp.minimum(count, WIN), _drain, 0)

    plsc.subcore_barrier()

    sl = NPAD // NS
    pltpu.sync_copy(agg_sh.at[pl.ds(sid * sl, sl)],
                    out_hbm.at[cid, pl.ds(sid * sl, sl)])


_sc_edge = functools.partial(
    pl.kernel,
    out_type=jax.ShapeDtypeStruct((NC, NPAD), jnp.float32),
    mesh=plsc.VectorSubcoreMesh(core_axis_name="c", subcore_axis_name="s"),
    scratch_types=[
        pltpu.VMEM((N,), jnp.float32),
        pltpu.VMEM((MAXCH, CHUNK), jnp.int32),
        pltpu.VMEM((MAXCH, CHUNK), jnp.int32),
        pltpu.VMEM((MAXCH * CHUNK,), jnp.float32),
        pltpu.VMEM((MAXCH, CHUNK), jnp.float32),
        pltpu.VMEM((NPAD // NS,), jnp.float32),
        pltpu.VMEM_SHARED((NPAD,), jnp.float32),
        pltpu.SemaphoreType.DMA,
        pltpu.SemaphoreType.DMA,
        pltpu.SemaphoreType.DMA,
        pltpu.SemaphoreType.DMA,
        pltpu.SemaphoreType.DMA,
    ],
    compiler_params=pltpu.CompilerParams(needs_layout_passes=False,
                                         use_tc_tiling_on_sc=False),
)(_sc_edge_body)


_RB = 1024
_RT = _RB // D


def _scale_body(x_ref, agg_ref, b_ref, eye_ref, o_ref):
    a = agg_ref[0] + agg_ref[1] + b_ref[0]
    attn = jnp.tanh(a)
    t = jax.lax.dot_general(eye_ref[...], attn, (((1,), (1,)), ((), ())),
                            preferred_element_type=jnp.float32,
                            precision=jax.lax.Precision.HIGHEST)
    for r in range(_RT):
        o_ref[pl.ds(r * D, D), :] = (
            x_ref[pl.ds(r * D, D), :] * t[:, r:r + 1])


def _scale(x, agg3, b, eye):
    grid = -(-N // _RB)
    return pl.pallas_call(
        _scale_body,
        grid=(grid,),
        in_specs=[
            pl.BlockSpec((_RB, D), lambda i: (i, 0)),
            pl.BlockSpec((NC, _RT, D), lambda i: (0, i, 0)),
            pl.BlockSpec(memory_space=pltpu.SMEM),
            pl.BlockSpec((D, D), lambda i: (0, 0)),
        ],
        out_specs=pl.BlockSpec((_RB, D), lambda i: (i, 0)),
        out_shape=jax.ShapeDtypeStruct((N, D), jnp.float32),
    )(x, agg3, b, eye)


def kernel(x, edge_index, edge_weight, W, b):
    eidx3 = edge_index.astype(jnp.int32).reshape(2, NCH, CHUNK)
    support = _support(x, W.reshape(1, D))
    agg2 = _sc_edge(support, eidx3, edge_weight)
    agg3 = agg2.reshape(NC, NPAD // D, D)
    return _scale(x, agg3, b, jnp.eye(D, dtype=jnp.float32))


# speedup vs baseline: 1.2725x; 1.0448x over previous
import functools

import jax
import jax.numpy as jnp
from jax import lax
from jax.experimental import pallas as pl
from jax.experimental.pallas import tpu as pltpu
from jax.experimental.pallas import tpu_sc as plsc

N = 10000
E = 320000
D = 128

NC = 2
NS = 16
NW = NC * NS
CHUNK = 128
NCH = E // CHUNK
CH_BASE = NCH // NW
CH_EXTRA = NCH % NW
MAXCH = CH_BASE + 1
NPAD = 10240
WIN = 16


_MB = 512


def _mv_body(wt_ref, x_ref, o_ref):
    o_ref[...] = jax.lax.dot_general(
        wt_ref[...], x_ref[...], (((1,), (1,)), ((), ())),
        preferred_element_type=jnp.float32)


def _support(x, Wt):
    return pl.pallas_call(
        _mv_body,
        out_shape=jax.ShapeDtypeStruct((1, N), jnp.float32),
    )(Wt, x)


def _sc_edge_body(support_hbm, eidx_hbm, ew_hbm, out_hbm,
                  support_v, src_v, dst_v, ew_v, msgs_v, zbuf_v, agg_sh,
                  sem_sup, sem_src, sem_dst, sem_ew, sem_scat):
    cid = lax.axis_index("c")
    sid = lax.axis_index("s")
    wid = sid * NC + cid

    base = wid * CH_BASE + jnp.minimum(wid, CH_EXTRA)
    has_extra = wid < CH_EXTRA
    count = CH_BASE + has_extra.astype(jnp.int32)

    c_sup = pltpu.async_copy(support_hbm.at[0], support_v, sem_sup)
    c_src = pltpu.async_copy(eidx_hbm.at[0, pl.ds(base, CH_BASE), :],
                             src_v.at[pl.ds(0, CH_BASE), :], sem_src)
    c_dst = pltpu.async_copy(eidx_hbm.at[1, pl.ds(base, CH_BASE), :],
                             dst_v.at[pl.ds(0, CH_BASE), :], sem_dst)
    c_ew = pltpu.async_copy(ew_hbm.at[pl.ds(base * CHUNK, CH_BASE * CHUNK)],
                            ew_v.at[pl.ds(0, CH_BASE * CHUNK)], sem_ew)

    @pl.when(has_extra)
    def _():
        pltpu.async_copy(eidx_hbm.at[0, pl.ds(base + CH_BASE, 1), :],
                         src_v.at[pl.ds(CH_BASE, 1), :], sem_src)
        pltpu.async_copy(eidx_hbm.at[1, pl.ds(base + CH_BASE, 1), :],
                         dst_v.at[pl.ds(CH_BASE, 1), :], sem_dst)
        pltpu.async_copy(
            ew_hbm.at[pl.ds((base + CH_BASE) * CHUNK, CHUNK)],
            ew_v.at[pl.ds(CH_BASE * CHUNK, CHUNK)], sem_ew)

    def _z(i, c):
        zbuf_v[pl.ds(i * 16, 16)] = jnp.zeros((16,), jnp.float32)
        return c
    lax.fori_loop(0, NPAD // NS // 16, _z, 0, unroll=True)
    pltpu.sync_copy(zbuf_v, agg_sh.at[pl.ds(sid * (NPAD // NS), NPAD // NS)])

    c_sup.wait()
    c_src.wait()
    c_dst.wait()
    c_ew.wait()

    @pl.when(has_extra)
    def _():
        pltpu.make_async_copy(eidx_hbm.at[0, pl.ds(base + CH_BASE, 1), :],
                              src_v.at[pl.ds(CH_BASE, 1), :], sem_src).wait()
        pltpu.make_async_copy(eidx_hbm.at[1, pl.ds(base + CH_BASE, 1), :],
                              dst_v.at[pl.ds(CH_BASE, 1), :], sem_dst).wait()
        pltpu.make_async_copy(
            ew_hbm.at[pl.ds((base + CH_BASE) * CHUNK, CHUNK)],
            ew_v.at[pl.ds(CH_BASE * CHUNK, CHUNK)], sem_ew).wait()

    plsc.subcore_barrier()

    def _chunk(j, carry):
        for g in range(CHUNK // 16):
            s16 = src_v[j, pl.ds(g * 16, 16)]
            vals = plsc.load_gather(support_v, [s16])
            w16 = ew_v[pl.ds(j * CHUNK + g * 16, 16)]
            msgs_v[j, pl.ds(g * 16, 16)] = vals * w16
        pltpu.async_copy(msgs_v.at[j], agg_sh.at[dst_v.at[j]], sem_scat,
                         add=True)

        @pl.when(j >= WIN)
        def _():
            pltpu.make_async_copy(msgs_v.at[0], agg_sh.at[dst_v.at[0]],
                                  sem_scat).wait()
        return carry

    lax.fori_loop(0, count, _chunk, 0)

    def _drain(j, carry):
        pltpu.make_async_copy(msgs_v.at[0], agg_sh.at[dst_v.at[0]],
                              sem_scat).wait()
        return carry
    lax.fori_loop(0, jnp.minimum(count, WIN), _drain, 0)

    plsc.subcore_barrier()

    sl = NPAD // NS
    pltpu.sync_copy(agg_sh.at[pl.ds(sid * sl, sl)],
                    out_hbm.at[cid, pl.ds(sid * sl, sl)])


_sc_edge = functools.partial(
    pl.kernel,
    out_type=jax.ShapeDtypeStruct((NC, NPAD), jnp.float32),
    mesh=plsc.VectorSubcoreMesh(core_axis_name="c", subcore_axis_name="s"),
    scratch_types=[
        pltpu.VMEM((N,), jnp.float32),
        pltpu.VMEM((MAXCH, CHUNK), jnp.int32),
        pltpu.VMEM((MAXCH, CHUNK), jnp.int32),
        pltpu.VMEM((MAXCH * CHUNK,), jnp.float32),
        pltpu.VMEM((MAXCH, CHUNK), jnp.float32),
        pltpu.VMEM((NPAD // NS,), jnp.float32),
        pltpu.VMEM_SHARED((NPAD,), jnp.float32),
        pltpu.SemaphoreType.DMA,
        pltpu.SemaphoreType.DMA,
        pltpu.SemaphoreType.DMA,
        pltpu.SemaphoreType.DMA,
        pltpu.SemaphoreType.DMA,
    ],
    compiler_params=pltpu.CompilerParams(needs_layout_passes=False,
                                         use_tc_tiling_on_sc=False),
)(_sc_edge_body)


_RB = 2048
_RT = _RB // D


def _scale_body(x_ref, agg_ref, b_ref, eye_ref, o_ref):
    a = agg_ref[0] + agg_ref[1] + b_ref[0]
    attn = jnp.tanh(a)
    t = jax.lax.dot_general(eye_ref[...], attn, (((1,), (1,)), ((), ())),
                            preferred_element_type=jnp.float32,
                            precision=jax.lax.Precision.HIGHEST)
    for r in range(_RT):
        o_ref[pl.ds(r * D, D), :] = (
            x_ref[pl.ds(r * D, D), :] * t[:, r:r + 1])


def _scale(x, agg3, b, eye):
    grid = -(-N // _RB)
    return pl.pallas_call(
        _scale_body,
        grid=(grid,),
        in_specs=[
            pl.BlockSpec((_RB, D), lambda i: (i, 0)),
            pl.BlockSpec((NC, _RT, D), lambda i: (0, i, 0)),
            pl.BlockSpec(memory_space=pltpu.SMEM),
            pl.BlockSpec((D, D), lambda i: (0, 0)),
        ],
        out_specs=pl.BlockSpec((_RB, D), lambda i: (i, 0)),
        out_shape=jax.ShapeDtypeStruct((N, D), jnp.float32),
    )(x, agg3, b, eye)


def kernel(x, edge_index, edge_weight, W, b):
    eidx3 = edge_index.astype(jnp.int32).reshape(2, NCH, CHUNK)
    support = _support(x, W.reshape(1, D))
    agg2 = _sc_edge(support, eidx3, edge_weight)
    agg3 = agg2.reshape(NC, NPAD // D, D)
    return _scale(x, agg3, b, jnp.eye(D, dtype=jnp.float32))
